# baseline (speedup 1.0000x reference)
import jax
import jax.numpy as jnp
from jax import lax
from jax.experimental import pallas as pl
from jax.experimental.pallas import tpu as pltpu

N_INPUTS = 4096
N_NEURONS = 4096
N_OUTPUTS = 1024
BATCH = 4096

TK1 = 256
TB = 512
K_TILES = N_INPUTS // TK1
B_TILES = BATCH // TB


def _body(wout_ref, w_ref, x_ref, out_ref, t_ref, wout_bf_ref):
    i = pl.program_id(0)

    @pl.when(i == 0)
    def _cast_wout():
        wout_bf_ref[...] = wout_ref[...].astype(jnp.bfloat16)

    @pl.when(i < K_TILES)
    def _build_t():
        acc = jnp.dot(wout_bf_ref[...],
                      w_ref[...].astype(jnp.bfloat16),
                      preferred_element_type=jnp.float32)
        t_ref[:, pl.ds(i * TK1, TK1)] = acc.astype(jnp.bfloat16)

    @pl.when(i >= K_TILES)
    def _emit_out():
        out_ref[...] = lax.dot_general(
            t_ref[...], x_ref[...].astype(jnp.bfloat16),
            dimension_numbers=(((1,), (1,)), ((), ())),
            preferred_element_type=jnp.float32)


@jax.jit
def kernel(x, w, wout):
    kmax = K_TILES - 1
    out = pl.pallas_call(
        _body,
        grid=(K_TILES + B_TILES,),
        in_specs=[
            pl.BlockSpec((N_OUTPUTS, N_NEURONS), lambda i: (0, 0)),
            pl.BlockSpec((N_NEURONS, TK1),
                         lambda i: (0, jnp.minimum(i, kmax))),
            pl.BlockSpec((TB, N_INPUTS),
                         lambda i: (jnp.maximum(i - K_TILES, 0), 0)),
        ],
        out_specs=pl.BlockSpec((N_OUTPUTS, TB),
                               lambda i: (0, jnp.maximum(i - K_TILES, 0))),
        out_shape=jax.ShapeDtypeStruct((N_OUTPUTS, BATCH), jnp.float32),
        scratch_shapes=[pltpu.VMEM((N_OUTPUTS, N_INPUTS), jnp.bfloat16),
                        pltpu.VMEM((N_OUTPUTS, N_NEURONS), jnp.bfloat16)],
        compiler_params=pltpu.CompilerParams(
            vmem_limit_bytes=100 * 1024 * 1024),
    )(wout, w, x)
    return out


# speedup vs baseline: 1.0120x; 1.0120x over previous
import jax
import jax.numpy as jnp
from jax import lax
from jax.experimental import pallas as pl
from jax.experimental.pallas import tpu as pltpu

N_INPUTS = 4096
N_NEURONS = 4096
N_OUTPUTS = 1024
BATCH = 4096

TK1 = 256
TB = 512
K_TILES = N_INPUTS // TK1
B_TILES = BATCH // TB


def _body(wout_ref, w_ref, x_ref, out_ref, t_ref):
    i = pl.program_id(0)

    @pl.when(i < K_TILES)
    def _build_t():
        acc = jnp.dot(wout_ref[...].astype(jnp.bfloat16),
                      w_ref[...].astype(jnp.bfloat16),
                      preferred_element_type=jnp.float32)
        t_ref[:, pl.ds(i * TK1, TK1)] = acc.astype(jnp.bfloat16)

    @pl.when(i >= K_TILES)
    def _emit_out():
        out_ref[...] = lax.dot_general(
            t_ref[...], x_ref[...].astype(jnp.bfloat16),
            dimension_numbers=(((1,), (1,)), ((), ())),
            preferred_element_type=jnp.float32)


@jax.jit
def kernel(x, w, wout):
    kmax = K_TILES - 1
    out = pl.pallas_call(
        _body,
        grid=(K_TILES + B_TILES,),
        in_specs=[
            pl.BlockSpec((N_OUTPUTS, N_NEURONS), lambda i: (0, 0)),
            pl.BlockSpec((N_NEURONS, TK1),
                         lambda i: (0, jnp.minimum(i, kmax))),
            pl.BlockSpec((TB, N_INPUTS),
                         lambda i: (jnp.maximum(i - K_TILES, 0), 0)),
        ],
        out_specs=pl.BlockSpec((N_OUTPUTS, TB),
                               lambda i: (0, jnp.maximum(i - K_TILES, 0))),
        out_shape=jax.ShapeDtypeStruct((N_OUTPUTS, BATCH), jnp.float32),
        scratch_shapes=[pltpu.VMEM((N_OUTPUTS, N_INPUTS), jnp.bfloat16)],
    )(wout, w, x)
    return out
